# baseline (speedup 1.0000x reference)
import jax
import jax.numpy as jnp
from jax import lax
from jax.experimental import pallas as pl
from jax.experimental.pallas import tpu as pltpu

D_MODEL = 2048
D_FF = 8192
NUM_OPS = 16

BM = 512
BF = 512


def _route_body(x_ref, op_ref):
    v = x_ref[...]
    mx = jnp.max(v, axis=1, keepdims=True)
    idx = lax.broadcasted_iota(jnp.int32, v.shape, 1)
    cand = jnp.where(v == mx, idx, NUM_OPS)
    op_ref[0] = jnp.min(cand)


def _ffn_body(op_ref, x_ref, w1_ref, b1_ref, w2_ref, b2_ref, o_ref):
    f = pl.program_id(1)
    h = jnp.dot(x_ref[...], w1_ref[0], preferred_element_type=jnp.float32)
    h = jnp.maximum(h + b1_ref[0], 0.0)
    contrib = jnp.dot(h, w2_ref[0], preferred_element_type=jnp.float32)

    @pl.when(f == 0)
    def _init():
        o_ref[...] = contrib

    @pl.when(f != 0)
    def _acc():
        o_ref[...] += contrib

    @pl.when(f == pl.num_programs(1) - 1)
    def _finish():
        o_ref[...] += b2_ref[0]


def kernel(x, W1, b1, W2, b2):
    batch, seq, d_model = x.shape
    m_total = batch * seq
    xf = x.reshape(m_total, d_model)

    op_arr = pl.pallas_call(
        _route_body,
        out_shape=jax.ShapeDtypeStruct((1,), jnp.int32),
        out_specs=pl.BlockSpec(memory_space=pltpu.SMEM),
    )(xf[0:1, :NUM_OPS])

    grid = (m_total // BM, D_FF // BF)
    b1r = b1.reshape(b1.shape[0], 1, D_FF)
    b2r = b2.reshape(b2.shape[0], 1, d_model)

    out = pl.pallas_call(
        _ffn_body,
        grid_spec=pltpu.PrefetchScalarGridSpec(
            num_scalar_prefetch=1,
            grid=grid,
            in_specs=[
                pl.BlockSpec((BM, d_model), lambda m, f, op: (m, 0)),
                pl.BlockSpec((1, d_model, BF), lambda m, f, op: (op[0], 0, f)),
                pl.BlockSpec((1, 1, BF), lambda m, f, op: (op[0], 0, f)),
                pl.BlockSpec((1, BF, d_model), lambda m, f, op: (op[0], f, 0)),
                pl.BlockSpec((1, 1, d_model), lambda m, f, op: (op[0], 0, 0)),
            ],
            out_specs=pl.BlockSpec((BM, d_model), lambda m, f, op: (m, 0)),
        ),
        out_shape=jax.ShapeDtypeStruct((m_total, d_model), jnp.float32),
        compiler_params=pltpu.CompilerParams(
            dimension_semantics=("parallel", "arbitrary"),
        ),
    )(op_arr, xf, W1, b1r, W2, b2r)

    return out.reshape(batch, seq, d_model)


# speedup vs baseline: 1.0454x; 1.0454x over previous
import jax
import jax.numpy as jnp
from jax import lax
from jax.experimental import pallas as pl
from jax.experimental.pallas import tpu as pltpu

D_MODEL = 2048
D_FF = 8192
NUM_OPS = 16

BM = 1024
BF = 512
CF = 512


def _route_body(x_ref, op_ref):
    v = x_ref[...]
    mx = jnp.max(v, axis=1, keepdims=True)
    idx = lax.broadcasted_iota(jnp.int32, v.shape, 1)
    cand = jnp.where(v == mx, idx, NUM_OPS)
    op_ref[0] = jnp.min(cand)


def _gather_cast_body(op_ref, w1_ref, w2_ref, w1b_ref, w2b_ref):
    w1b_ref[...] = w1_ref[0].astype(jnp.bfloat16)
    w2b_ref[...] = w2_ref[0].astype(jnp.bfloat16)


def _ffn_body(op_ref, x_ref, w1_ref, b1_ref, w2_ref, b2_ref, o_ref):
    f = pl.program_id(1)
    h = jnp.dot(x_ref[...], w1_ref[...], preferred_element_type=jnp.float32)
    h = jnp.maximum(h + b1_ref[0], 0.0).astype(jnp.bfloat16)
    contrib = jnp.dot(h, w2_ref[...], preferred_element_type=jnp.float32)

    @pl.when(f == 0)
    def _init():
        o_ref[...] = contrib

    @pl.when(f != 0)
    def _acc():
        o_ref[...] += contrib

    @pl.when(f == pl.num_programs(1) - 1)
    def _finish():
        o_ref[...] += b2_ref[0]


def kernel(x, W1, b1, W2, b2):
    batch, seq, d_model = x.shape
    m_total = batch * seq
    xf = x.reshape(m_total, d_model)

    op_arr = pl.pallas_call(
        _route_body,
        out_shape=jax.ShapeDtypeStruct((1,), jnp.int32),
        out_specs=pl.BlockSpec(memory_space=pltpu.SMEM),
    )(xf[0:1, :NUM_OPS])

    w1b, w2b = pl.pallas_call(
        _gather_cast_body,
        grid_spec=pltpu.PrefetchScalarGridSpec(
            num_scalar_prefetch=1,
            grid=(D_FF // CF,),
            in_specs=[
                pl.BlockSpec((1, d_model, CF), lambda f, op: (op[0], 0, f)),
                pl.BlockSpec((1, CF, d_model), lambda f, op: (op[0], f, 0)),
            ],
            out_specs=[
                pl.BlockSpec((d_model, CF), lambda f, op: (0, f)),
                pl.BlockSpec((CF, d_model), lambda f, op: (f, 0)),
            ],
        ),
        out_shape=[
            jax.ShapeDtypeStruct((d_model, D_FF), jnp.bfloat16),
            jax.ShapeDtypeStruct((D_FF, d_model), jnp.bfloat16),
        ],
        compiler_params=pltpu.CompilerParams(
            dimension_semantics=("arbitrary",),
        ),
    )(op_arr, W1, W2)

    xb = xf.astype(jnp.bfloat16)
    b1r = b1.reshape(b1.shape[0], 1, D_FF)
    b2r = b2.reshape(b2.shape[0], 1, d_model)

    grid = (m_total // BM, D_FF // BF)

    out = pl.pallas_call(
        _ffn_body,
        grid_spec=pltpu.PrefetchScalarGridSpec(
            num_scalar_prefetch=1,
            grid=grid,
            in_specs=[
                pl.BlockSpec((BM, d_model), lambda m, f, op: (m, 0)),
                pl.BlockSpec((d_model, BF), lambda m, f, op: (0, f)),
                pl.BlockSpec((1, 1, BF), lambda m, f, op: (op[0], 0, f)),
                pl.BlockSpec((BF, d_model), lambda m, f, op: (f, 0)),
                pl.BlockSpec((1, 1, d_model), lambda m, f, op: (op[0], 0, 0)),
            ],
            out_specs=pl.BlockSpec((BM, d_model), lambda m, f, op: (m, 0)),
        ),
        out_shape=jax.ShapeDtypeStruct((m_total, d_model), jnp.float32),
        compiler_params=pltpu.CompilerParams(
            dimension_semantics=("parallel", "arbitrary"),
        ),
    )(op_arr, xb, w1b, b1r, w2b, b2r)

    return out.reshape(batch, seq, d_model)


# speedup vs baseline: 1.1521x; 1.1021x over previous
import jax
import jax.numpy as jnp
from jax import lax
from jax.experimental import pallas as pl
from jax.experimental.pallas import tpu as pltpu

D_MODEL = 2048
D_FF = 8192
NUM_OPS = 16

BM = 1024
BF = 512
BN = 256
NF = D_FF // BF
NN = D_MODEL // BN
CF = 512


def _route_body(x_ref, op_ref):
    v = x_ref[...]
    mx = jnp.max(v, axis=1, keepdims=True)
    idx = lax.broadcasted_iota(jnp.int32, v.shape, 1)
    cand = jnp.where(v == mx, idx, NUM_OPS)
    op_ref[0] = jnp.min(cand)


def _gather_cast_body(op_ref, w1_ref, w2_ref, w1b_ref, w2b_ref):
    w1b_ref[...] = w1_ref[0].astype(jnp.bfloat16)
    w2b_ref[...] = w2_ref[0].astype(jnp.bfloat16)


def _ffn_body(op_ref, x_ref, w1_ref, b1_ref, w2_ref, b2_ref, o_ref, h_ref):
    j = pl.program_id(1)

    @pl.when(j < NF)
    def _hidden():
        h = jnp.dot(x_ref[...], w1_ref[...], preferred_element_type=jnp.float32)
        h = jnp.maximum(h + b1_ref[0], 0.0)
        h_ref[:, pl.ds(j * BF, BF)] = h.astype(jnp.bfloat16)

    @pl.when(j >= NF)
    def _output():
        o_ref[...] = (
            jnp.dot(h_ref[...], w2_ref[...], preferred_element_type=jnp.float32)
            + b2_ref[0]
        )


def kernel(x, W1, b1, W2, b2):
    batch, seq, d_model = x.shape
    m_total = batch * seq
    xf = x.reshape(m_total, d_model)

    op_arr = pl.pallas_call(
        _route_body,
        out_shape=jax.ShapeDtypeStruct((1,), jnp.int32),
        out_specs=pl.BlockSpec(memory_space=pltpu.SMEM),
    )(xf[0:1, :NUM_OPS])

    w1b, w2b = pl.pallas_call(
        _gather_cast_body,
        grid_spec=pltpu.PrefetchScalarGridSpec(
            num_scalar_prefetch=1,
            grid=(D_FF // CF,),
            in_specs=[
                pl.BlockSpec((1, d_model, CF), lambda f, op: (op[0], 0, f)),
                pl.BlockSpec((1, CF, d_model), lambda f, op: (op[0], f, 0)),
            ],
            out_specs=[
                pl.BlockSpec((d_model, CF), lambda f, op: (0, f)),
                pl.BlockSpec((CF, d_model), lambda f, op: (f, 0)),
            ],
        ),
        out_shape=[
            jax.ShapeDtypeStruct((d_model, D_FF), jnp.bfloat16),
            jax.ShapeDtypeStruct((D_FF, d_model), jnp.bfloat16),
        ],
        compiler_params=pltpu.CompilerParams(
            dimension_semantics=("arbitrary",),
        ),
    )(op_arr, W1, W2)

    xb = xf.astype(jnp.bfloat16)
    b1r = b1.reshape(b1.shape[0], 1, D_FF)
    b2r = b2.reshape(b2.shape[0], 1, d_model)

    grid = (m_total // BM, NF + NN)

    out = pl.pallas_call(
        _ffn_body,
        grid_spec=pltpu.PrefetchScalarGridSpec(
            num_scalar_prefetch=1,
            grid=grid,
            in_specs=[
                pl.BlockSpec((BM, d_model), lambda m, j, op: (m, 0)),
                pl.BlockSpec(
                    (d_model, BF),
                    lambda m, j, op: (0, jnp.minimum(j, NF - 1)),
                ),
                pl.BlockSpec(
                    (1, 1, BF),
                    lambda m, j, op: (op[0], 0, jnp.minimum(j, NF - 1)),
                ),
                pl.BlockSpec(
                    (D_FF, BN),
                    lambda m, j, op: (0, jnp.maximum(j - NF, 0)),
                ),
                pl.BlockSpec(
                    (1, 1, BN),
                    lambda m, j, op: (op[0], 0, jnp.maximum(j - NF, 0)),
                ),
            ],
            out_specs=pl.BlockSpec(
                (BM, BN),
                lambda m, j, op: (m, jnp.maximum(j - NF, 0)),
            ),
            scratch_shapes=[pltpu.VMEM((BM, D_FF), jnp.bfloat16)],
        ),
        out_shape=jax.ShapeDtypeStruct((m_total, d_model), jnp.float32),
        compiler_params=pltpu.CompilerParams(
            dimension_semantics=("parallel", "arbitrary"),
        ),
    )(op_arr, xb, w1b, b1r, w2b, b2r)

    return out.reshape(batch, seq, d_model)


# speedup vs baseline: 1.2170x; 1.0564x over previous
import jax
import jax.numpy as jnp
from jax import lax
from jax.experimental import pallas as pl
from jax.experimental.pallas import tpu as pltpu

D_MODEL = 2048
D_FF = 8192
NUM_OPS = 16

BM = 1024
BF = 1024
BN = 512
NF = D_FF // BF
NN = D_MODEL // BN
CF = 512


def _route_body(x_ref, op_ref):
    v = x_ref[...]
    mx = jnp.max(v, axis=1, keepdims=True)
    idx = lax.broadcasted_iota(jnp.int32, v.shape, 1)
    cand = jnp.where(v == mx, idx, NUM_OPS)
    op_ref[0] = jnp.min(cand)


def _gather_cast_body(op_ref, w1_ref, w2_ref, w1b_ref, w2b_ref):
    w1b_ref[...] = w1_ref[0].astype(jnp.bfloat16)
    w2b_ref[...] = w2_ref[0].astype(jnp.bfloat16)


def _ffn_body(op_ref, x_ref, w1_ref, b1_ref, w2_ref, b2_ref, o_ref, h_ref):
    j = pl.program_id(1)

    @pl.when(j < NF)
    def _hidden():
        h = jnp.dot(x_ref[...], w1_ref[...], preferred_element_type=jnp.float32)
        h = jnp.maximum(h + b1_ref[0], 0.0)
        h_ref[:, pl.ds(j * BF, BF)] = h.astype(jnp.bfloat16)

    @pl.when(j >= NF)
    def _output():
        o_ref[...] = (
            jnp.dot(h_ref[...], w2_ref[...], preferred_element_type=jnp.float32)
            + b2_ref[0]
        )


def kernel(x, W1, b1, W2, b2):
    batch, seq, d_model = x.shape
    m_total = batch * seq
    xf = x.reshape(m_total, d_model)

    op_arr = pl.pallas_call(
        _route_body,
        out_shape=jax.ShapeDtypeStruct((1,), jnp.int32),
        out_specs=pl.BlockSpec(memory_space=pltpu.SMEM),
    )(xf[0:1, :NUM_OPS])

    w1b, w2b = pl.pallas_call(
        _gather_cast_body,
        grid_spec=pltpu.PrefetchScalarGridSpec(
            num_scalar_prefetch=1,
            grid=(D_FF // CF,),
            in_specs=[
                pl.BlockSpec((1, d_model, CF), lambda f, op: (op[0], 0, f)),
                pl.BlockSpec((1, CF, d_model), lambda f, op: (op[0], f, 0)),
            ],
            out_specs=[
                pl.BlockSpec((d_model, CF), lambda f, op: (0, f)),
                pl.BlockSpec((CF, d_model), lambda f, op: (f, 0)),
            ],
        ),
        out_shape=[
            jax.ShapeDtypeStruct((d_model, D_FF), jnp.bfloat16),
            jax.ShapeDtypeStruct((D_FF, d_model), jnp.bfloat16),
        ],
        compiler_params=pltpu.CompilerParams(
            dimension_semantics=("arbitrary",),
        ),
    )(op_arr, W1, W2)

    xb = xf.astype(jnp.bfloat16)
    b1r = b1.reshape(b1.shape[0], 1, D_FF)
    b2r = b2.reshape(b2.shape[0], 1, d_model)

    grid = (m_total // BM, NF + NN)

    out = pl.pallas_call(
        _ffn_body,
        grid_spec=pltpu.PrefetchScalarGridSpec(
            num_scalar_prefetch=1,
            grid=grid,
            in_specs=[
                pl.BlockSpec((BM, d_model), lambda m, j, op: (m, 0)),
                pl.BlockSpec(
                    (d_model, BF),
                    lambda m, j, op: (0, jnp.minimum(j, NF - 1)),
                ),
                pl.BlockSpec(
                    (1, 1, BF),
                    lambda m, j, op: (op[0], 0, jnp.minimum(j, NF - 1)),
                ),
                pl.BlockSpec(
                    (D_FF, BN),
                    lambda m, j, op: (0, jnp.maximum(j - NF, 0)),
                ),
                pl.BlockSpec(
                    (1, 1, BN),
                    lambda m, j, op: (op[0], 0, jnp.maximum(j - NF, 0)),
                ),
            ],
            out_specs=pl.BlockSpec(
                (BM, BN),
                lambda m, j, op: (m, jnp.maximum(j - NF, 0)),
            ),
            scratch_shapes=[pltpu.VMEM((BM, D_FF), jnp.bfloat16)],
        ),
        out_shape=jax.ShapeDtypeStruct((m_total, d_model), jnp.float32),
        compiler_params=pltpu.CompilerParams(
            dimension_semantics=("parallel", "arbitrary"),
        ),
    )(op_arr, xb, w1b, b1r, w2b, b2r)

    return out.reshape(batch, seq, d_model)


# speedup vs baseline: 1.2721x; 1.0452x over previous
import jax
import jax.numpy as jnp
from jax import lax
from jax.experimental import pallas as pl
from jax.experimental.pallas import tpu as pltpu

D_MODEL = 2048
D_FF = 8192
NUM_OPS = 16

BM = 1024
BF = 512
BN = 256
NF = D_FF // BF
NN = D_MODEL // BN


def _route_body(x_ref, op_ref):
    v = x_ref[...]
    mx = jnp.max(v, axis=1, keepdims=True)
    idx = lax.broadcasted_iota(jnp.int32, v.shape, 1)
    cand = jnp.where(v == mx, idx, NUM_OPS)
    op_ref[0] = jnp.min(cand)


def _ffn_body(op_ref, x_ref, w1_ref, b1_ref, w2_ref, b2_ref, o_ref, h_ref):
    j = pl.program_id(1)

    @pl.when(j < NF)
    def _hidden():
        h = jnp.dot(x_ref[...], w1_ref[0], preferred_element_type=jnp.float32)
        h = jnp.maximum(h + b1_ref[0], 0.0)
        h_ref[:, pl.ds(j * BF, BF)] = h.astype(jnp.bfloat16)

    @pl.when(j >= NF)
    def _output():
        w2b = w2_ref[0].astype(jnp.bfloat16)
        o_ref[...] = (
            jnp.dot(h_ref[...], w2b, preferred_element_type=jnp.float32)
            + b2_ref[0]
        )


def kernel(x, W1, b1, W2, b2):
    batch, seq, d_model = x.shape
    m_total = batch * seq
    xf = x.reshape(m_total, d_model)

    op_arr = pl.pallas_call(
        _route_body,
        out_shape=jax.ShapeDtypeStruct((1,), jnp.int32),
        out_specs=pl.BlockSpec(memory_space=pltpu.SMEM),
    )(xf[0:1, :NUM_OPS])

    b1r = b1.reshape(b1.shape[0], 1, D_FF)
    b2r = b2.reshape(b2.shape[0], 1, d_model)

    grid = (m_total // BM, NF + NN)

    out = pl.pallas_call(
        _ffn_body,
        grid_spec=pltpu.PrefetchScalarGridSpec(
            num_scalar_prefetch=1,
            grid=grid,
            in_specs=[
                pl.BlockSpec((BM, d_model), lambda m, j, op: (m, 0)),
                pl.BlockSpec(
                    (1, d_model, BF),
                    lambda m, j, op: (op[0], 0, jnp.minimum(j, NF - 1)),
                ),
                pl.BlockSpec(
                    (1, 1, BF),
                    lambda m, j, op: (op[0], 0, jnp.minimum(j, NF - 1)),
                ),
                pl.BlockSpec(
                    (1, D_FF, BN),
                    lambda m, j, op: (op[0], 0, jnp.maximum(j - NF, 0)),
                ),
                pl.BlockSpec(
                    (1, 1, BN),
                    lambda m, j, op: (op[0], 0, jnp.maximum(j - NF, 0)),
                ),
            ],
            out_specs=pl.BlockSpec(
                (BM, BN),
                lambda m, j, op: (m, jnp.maximum(j - NF, 0)),
            ),
            scratch_shapes=[pltpu.VMEM((BM, D_FF), jnp.bfloat16)],
        ),
        out_shape=jax.ShapeDtypeStruct((m_total, d_model), jnp.float32),
        compiler_params=pltpu.CompilerParams(
            dimension_semantics=("parallel", "arbitrary"),
        ),
    )(op_arr, xf, W1, b1r, W2, b2r)

    return out.reshape(batch, seq, d_model)
